# baseline (speedup 1.0000x reference)
import numpy as np
import jax
import jax.numpy as jnp
from jax.experimental import pallas as pl
from jax.experimental.pallas import tpu as pltpu

LANES = 128
_TB = 512


def _conv1_sel():
    S1 = np.full((128, 512), -1, np.int64)
    for ph in range(4):
        di, dj = divmod(ph, 2)
        for q in range(14):
            for c in range(6):
                n0 = ph * 128 + (q + 1) * 6 + c
                for ki in range(3):
                    for kj in range(3):
                        k = (ki + di) * 30 + (kj + dj) + 2 * q
                        S1[k, n0] = ki * 3 + kj
    mid = np.array([[S1[r * 30 + j + 1] for j in range(28)]
                    for r in range(4)]).reshape(112, 512)
    top = np.array([[S1[(r + 1) * 30 + j + 1] for j in range(28)]
                    for r in range(3)]).reshape(84, 512)
    bot = np.array([[S1[r * 30 + j + 1] for j in range(28)]
                    for r in range(3)]).reshape(84, 512)
    sel = np.full((3, 176, 1024), -1, np.int64)
    sel[0, 0:84, 0:512] = top
    sel[0, 28:140, 512:1024] = mid
    sel[1, 0:112, 0:512] = mid
    sel[1, 56:168, 512:1024] = mid
    sel[2, 36:148, 0:512] = mid
    sel[2, 92:176, 512:1024] = bot
    A = (sel[None] == np.arange(9)[:, None, None, None])
    C = np.zeros((6, 1024), np.float32)
    for half in range(2):
        for ph in range(4):
            for q in range(14):
                for c in range(6):
                    C[c, half * 512 + ph * 128 + (q + 1) * 6 + c] = 1.0
    return A.astype(np.float32), C


def _conv2_sel():
    B = np.zeros((9 * 256, 54), np.float32)
    for t in range(9):
        for h in range(2):
            for k in range(96):
                B[t * 256 + h * 128 + k, t * 6 + k % 6] = 1.0
    O = np.zeros((128, 512), np.float32)
    for n in range(512):
        if n % 128 < 112:
            O[n % 16, n] = 1.0
    A = np.zeros((9, 2, 256, 512), np.float32)
    for r in range(4):
        p, h = divmod(r, 2)
        for ph in range(4):
            di, dj = divmod(ph, 2)
            ki = r - di
            if not 0 <= ki <= 2:
                continue
            for q2 in range(7):
                for kj in range(3):
                    cp = 2 * q2 + dj + kj
                    for ch in range(6):
                        kk = h * 128 + cp * 6 + ch
                        n0 = ph * 128 + q2 * 16
                        A[ki * 3 + kj, p, kk, n0:n0 + 16] = 1.0
    return B, O, A


def _bias_sel():
    CB1 = np.zeros((8, 128), np.float32)
    for q in range(14):
        for c in range(6):
            CB1[c, (q + 1) * 6 + c] = 1.0
    CB2 = np.zeros((16, 128), np.float32)
    for n in range(112):
        CB2[n % 16, n] = 1.0
    return CB1, CB2


_A1np, _C1np = _conv1_sel()
_B2np, _O2np, _A2np = _conv2_sel()
_CB1np, _CB2np = _bias_sel()


def _net_kernel(x_ref, a1, c1, b2c, o2c, a2, cb1, cb2,
                c1w, c2w, c1b, c2b, fw1, fw2, fw3, fb1, fb2, fb3,
                out, w1s, w2s, f1s, b1s, b2s):
    i = pl.program_id(0)
    tb = out.shape[0]

    @pl.when(i == 0)
    def _build():
        g1 = jnp.dot(c1w[:, 0:6], c1[...],
                     preferred_element_type=jnp.float32).astype(jnp.bfloat16)
        for j in range(3):
            acc = a1[0, j] * g1[0:1, :]
            for t in range(1, 9):
                acc = acc + a1[t, j] * g1[t:t + 1, :]
            w1s[j] = acc.astype(jnp.bfloat16)
        q = jnp.dot(b2c[...], c2w[...], preferred_element_type=jnp.float32)
        qn = jnp.dot(q.astype(jnp.bfloat16), o2c[...],
                     preferred_element_type=jnp.float32)
        qn = qn.astype(jnp.bfloat16).reshape(9, 256, 512)
        for p in range(2):
            acc = a2[0, p] * qn[0]
            for t in range(1, 9):
                acc = acc + a2[t, p] * qn[t]
            w2s[p] = acc.astype(jnp.bfloat16)
        f1s[...] = jnp.zeros(f1s.shape, f1s.dtype)
        for j in range(3):
            f1s[j, 0:112, :] = fw1[224 * j:224 * j + 112, :]
            f1s[j, 128:240, :] = fw1[224 * j + 112:224 * j + 224, :]
        f1s[3, 0:112, :] = fw1[672:784, :]
        acc1 = c1b[0:1, 0:1] * cb1[0:1, :]
        for c in range(1, 6):
            acc1 = acc1 + c1b[0:1, c:c + 1] * cb1[c:c + 1, :]
        b1s[...] = acc1
        acc2 = c2b[0:1, 0:1] * cb2[0:1, :]
        for o in range(1, 16):
            acc2 = acc2 + c2b[0:1, o:o + 1] * cb2[o:o + 1, :]
        b2s[...] = acc2

    xv = x_ref[...].reshape(tb, 784).astype(jnp.bfloat16)
    b1v = b1s[...]

    rows = [None] * 16
    zero = jnp.zeros((tb, 128), jnp.bfloat16)
    rows[0] = zero
    rows[15] = zero
    for j in range(7):
        po = 2 * j
        if j == 0:
            v = jnp.dot(xv[:, 0:176], w1s[0],
                        preferred_element_type=jnp.float32)
        elif j == 6:
            v = jnp.dot(xv[:, 608:784], w1s[2],
                        preferred_element_type=jnp.float32)
        else:
            v = jnp.dot(xv[:, 56 * po - 28:56 * po + 148], w1s[1],
                        preferred_element_type=jnp.float32)
        for s in range(2):
            b = 512 * s
            m = jnp.maximum(
                jnp.maximum(v[:, b:b + 128], v[:, b + 128:b + 256]),
                jnp.maximum(v[:, b + 256:b + 384], v[:, b + 384:b + 512]))
            rows[po + s + 1] = jnp.maximum(m + b1v, 0.0).astype(jnp.bfloat16)

    pairs = [jnp.concatenate([rows[2 * j], rows[2 * j + 1]], axis=1)
             for j in range(8)]

    b2v = b2s[...]
    row2s = []
    for po2 in range(7):
        acc = (jnp.dot(pairs[po2], w2s[0], preferred_element_type=jnp.float32)
               + jnp.dot(pairs[po2 + 1], w2s[1],
                         preferred_element_type=jnp.float32))
        m = jnp.maximum(jnp.maximum(acc[:, 0:128], acc[:, 128:256]),
                        jnp.maximum(acc[:, 256:384], acc[:, 384:512]))
        row2s.append(jnp.maximum(m + b2v, 0.0).astype(jnp.bfloat16))

    h1 = None
    for j in range(3):
        op = jnp.concatenate([row2s[2 * j], row2s[2 * j + 1]], axis=1)
        t = jnp.dot(op, f1s[j], preferred_element_type=jnp.float32)
        h1 = t if h1 is None else h1 + t
    h1 = h1 + jnp.dot(row2s[6], f1s[3][:128],
                      preferred_element_type=jnp.float32)

    h1 = jnp.maximum(h1 + fb1[...], 0.0).astype(jnp.bfloat16)
    h2 = jnp.maximum(
        jnp.dot(h1, fw2[...], preferred_element_type=jnp.float32) + fb2[...],
        0.0).astype(jnp.bfloat16)
    logits = jnp.dot(h2, fw3[...], preferred_element_type=jnp.float32) + fb3[...]
    out[...] = logits[:, :10]


def kernel(x, conv1_w, conv1_b, conv2_w, conv2_b,
           fc_w1, fc_b1, fc_w2, fc_b2, fc_w3, fc_b3):
    B = x.shape[0]
    tb = _TB if B % _TB == 0 else B
    grid = B // tb

    bf = jnp.bfloat16
    A1 = jnp.asarray(_A1np, bf)
    C1 = jnp.asarray(_C1np, bf)
    B2 = jnp.asarray(_B2np, bf)
    O2 = jnp.asarray(_O2np, bf)
    A2 = jnp.asarray(_A2np, bf)
    CB1 = jnp.asarray(_CB1np, jnp.float32)
    CB2 = jnp.asarray(_CB2np, jnp.float32)

    res = pl.pallas_call(
        _net_kernel,
        out_shape=jax.ShapeDtypeStruct((B, 10), jnp.float32),
        grid=(grid,),
        in_specs=[
            pl.BlockSpec((tb, 1, 28, 28), lambda i: (i, 0, 0, 0)),
            pl.BlockSpec((9, 3, 176, 1024), lambda i: (0, 0, 0, 0)),
            pl.BlockSpec((6, 1024), lambda i: (0, 0)),
            pl.BlockSpec((2304, 54), lambda i: (0, 0)),
            pl.BlockSpec((128, 512), lambda i: (0, 0)),
            pl.BlockSpec((9, 2, 256, 512), lambda i: (0, 0, 0, 0)),
            pl.BlockSpec((8, 128), lambda i: (0, 0)),
            pl.BlockSpec((16, 128), lambda i: (0, 0)),
            pl.BlockSpec((9, 128), lambda i: (0, 0)),
            pl.BlockSpec((54, 128), lambda i: (0, 0)),
            pl.BlockSpec((1, 128), lambda i: (0, 0)),
            pl.BlockSpec((1, 128), lambda i: (0, 0)),
            pl.BlockSpec((784, 128), lambda i: (0, 0)),
            pl.BlockSpec((128, 128), lambda i: (0, 0)),
            pl.BlockSpec((128, 128), lambda i: (0, 0)),
            pl.BlockSpec((1, 128), lambda i: (0, 0)),
            pl.BlockSpec((1, 128), lambda i: (0, 0)),
            pl.BlockSpec((1, 128), lambda i: (0, 0)),
        ],
        out_specs=pl.BlockSpec((tb, 10), lambda i: (i, 0)),
        scratch_shapes=[
            pltpu.VMEM((3, 176, 1024), bf),
            pltpu.VMEM((2, 256, 512), bf),
            pltpu.VMEM((4, 256, 128), bf),
            pltpu.VMEM((1, 128), jnp.float32),
            pltpu.VMEM((1, 128), jnp.float32),
        ],
        compiler_params=pltpu.CompilerParams(
            dimension_semantics=("arbitrary",)),
    )(x, A1, C1, B2, O2, A2, CB1, CB2,
      conv1_w, conv2_w, conv1_b, conv2_b, fc_w1, fc_w2, fc_w3,
      fc_b1, fc_b2, fc_b3)

    return res


# speedup vs baseline: 1.5152x; 1.5152x over previous
import numpy as np
import jax
import jax.numpy as jnp
from jax.experimental import pallas as pl
from jax.experimental.pallas import tpu as pltpu

LANES = 128
_TB = 512


def _conv1_sel():
    S1 = np.full((128, 512), -1, np.int64)
    for ph in range(4):
        di, dj = divmod(ph, 2)
        for q in range(14):
            for c in range(6):
                n0 = ph * 128 + (q + 1) * 6 + c
                for ki in range(3):
                    for kj in range(3):
                        k = (ki + di) * 30 + (kj + dj) + 2 * q
                        S1[k, n0] = ki * 3 + kj
    mid = np.array([[S1[r * 30 + j + 1] for j in range(28)]
                    for r in range(4)]).reshape(112, 512)
    top = np.array([[S1[(r + 1) * 30 + j + 1] for j in range(28)]
                    for r in range(3)]).reshape(84, 512)
    bot = np.array([[S1[r * 30 + j + 1] for j in range(28)]
                    for r in range(3)]).reshape(84, 512)
    sel = np.full((3, 176, 1024), -1, np.int64)
    sel[0, 0:84, 0:512] = top
    sel[0, 28:140, 512:1024] = mid
    sel[1, 0:112, 0:512] = mid
    sel[1, 56:168, 512:1024] = mid
    sel[2, 36:148, 0:512] = mid
    sel[2, 92:176, 512:1024] = bot
    A = (sel[None] == np.arange(9)[:, None, None, None])
    C = np.zeros((6, 1024), np.float32)
    for half in range(2):
        for ph in range(4):
            for q in range(14):
                for c in range(6):
                    C[c, half * 512 + ph * 128 + (q + 1) * 6 + c] = 1.0
    return A.astype(np.float32), C


def _conv2_sel():
    B = np.zeros((9 * 256, 54), np.float32)
    for t in range(9):
        for h in range(2):
            for k in range(96):
                B[t * 256 + h * 128 + k, t * 6 + k % 6] = 1.0
    O = np.zeros((128, 512), np.float32)
    for n in range(512):
        if n % 128 < 112:
            O[n % 16, n] = 1.0
    A = np.zeros((9, 2, 256, 512), np.float32)
    for r in range(4):
        p, h = divmod(r, 2)
        for ph in range(4):
            di, dj = divmod(ph, 2)
            ki = r - di
            if not 0 <= ki <= 2:
                continue
            for q2 in range(7):
                for kj in range(3):
                    cp = 2 * q2 + dj + kj
                    for ch in range(6):
                        kk = h * 128 + cp * 6 + ch
                        n0 = ph * 128 + q2 * 16
                        A[ki * 3 + kj, p, kk, n0:n0 + 16] = 1.0
    return B, O, A


def _bias_sel():
    CB1 = np.zeros((8, 128), np.float32)
    for q in range(14):
        for c in range(6):
            CB1[c, (q + 1) * 6 + c] = 1.0
    CB2 = np.zeros((16, 128), np.float32)
    for n in range(112):
        CB2[n % 16, n] = 1.0
    return CB1, CB2


_A1np, _C1np = _conv1_sel()
_B2np, _O2np, _A2np = _conv2_sel()
_CB1np, _CB2np = _bias_sel()


def _net_kernel(x_ref, a1, c1, b2c, o2c, a2, cb1, cb2,
                c1w, c2w, c1b, c2b, fw1, fw2, fw3, fb1, fb2, fb3,
                out, w1s, w2s, f1s, b1s, b2s):
    i = pl.program_id(0)
    tb = out.shape[0]

    @pl.when(i == 0)
    def _build():
        g1 = jnp.dot(c1w[:, 0:6], c1[...],
                     preferred_element_type=jnp.float32).astype(jnp.bfloat16)
        for j in range(3):
            acc = a1[0, j] * g1[0:1, :]
            for t in range(1, 9):
                acc = acc + a1[t, j] * g1[t:t + 1, :]
            w1s[j] = acc.astype(jnp.bfloat16)
        q = jnp.dot(b2c[...], c2w[...], preferred_element_type=jnp.float32)
        qn = jnp.dot(q.astype(jnp.bfloat16), o2c[...],
                     preferred_element_type=jnp.float32)
        qn = qn.astype(jnp.bfloat16).reshape(9, 256, 512)
        for p in range(2):
            acc = a2[0, p] * qn[0]
            for t in range(1, 9):
                acc = acc + a2[t, p] * qn[t]
            w2s[256 * p:256 * (p + 1)] = acc.astype(jnp.bfloat16)
        f1s[...] = jnp.zeros(f1s.shape, f1s.dtype)
        for j in range(3):
            f1s[j, 0:112, :] = fw1[224 * j:224 * j + 112, :]
            f1s[j, 128:240, :] = fw1[224 * j + 112:224 * j + 224, :]
        f1s[3, 0:112, :] = fw1[672:784, :]
        acc1 = c1b[0:1, 0:1] * cb1[0:1, :]
        for c in range(1, 6):
            acc1 = acc1 + c1b[0:1, c:c + 1] * cb1[c:c + 1, :]
        b1s[...] = acc1
        acc2 = c2b[0:1, 0:1] * cb2[0:1, :]
        for o in range(1, 16):
            acc2 = acc2 + c2b[0:1, o:o + 1] * cb2[o:o + 1, :]
        b2s[...] = acc2

    xv = x_ref[...].reshape(tb, 784).astype(jnp.bfloat16)
    b1v = b1s[...]

    rows = [None] * 16
    zero = jnp.zeros((tb, 128), jnp.bfloat16)
    rows[0] = zero
    rows[15] = zero
    for j in range(7):
        po = 2 * j
        if j == 0:
            v = jnp.dot(xv[:, 0:176], w1s[0],
                        preferred_element_type=jnp.float32)
        elif j == 6:
            v = jnp.dot(xv[:, 608:784], w1s[2],
                        preferred_element_type=jnp.float32)
        else:
            v = jnp.dot(xv[:, 56 * po - 28:56 * po + 148], w1s[1],
                        preferred_element_type=jnp.float32)
        for s in range(2):
            b = 512 * s
            m = jnp.maximum(
                jnp.maximum(v[:, b:b + 128], v[:, b + 128:b + 256]),
                jnp.maximum(v[:, b + 256:b + 384], v[:, b + 384:b + 512]))
            rows[po + s + 1] = jnp.maximum(m + b1v, 0.0).astype(jnp.bfloat16)

    b2v = b2s[...]
    w2v = w2s[...]
    row2s = []
    for po2 in range(7):
        quad = jnp.concatenate([rows[2 * po2 + r] for r in range(4)], axis=1)
        acc = jnp.dot(quad, w2v, preferred_element_type=jnp.float32)
        m = jnp.maximum(jnp.maximum(acc[:, 0:128], acc[:, 128:256]),
                        jnp.maximum(acc[:, 256:384], acc[:, 384:512]))
        row2s.append(jnp.maximum(m + b2v, 0.0).astype(jnp.bfloat16))

    h1 = None
    for j in range(3):
        op = jnp.concatenate([row2s[2 * j], row2s[2 * j + 1]], axis=1)
        t = jnp.dot(op, f1s[j], preferred_element_type=jnp.float32)
        h1 = t if h1 is None else h1 + t
    h1 = h1 + jnp.dot(row2s[6], f1s[3][:128],
                      preferred_element_type=jnp.float32)

    h1 = jnp.maximum(h1 + fb1[...], 0.0).astype(jnp.bfloat16)
    h2 = jnp.maximum(
        jnp.dot(h1, fw2[...], preferred_element_type=jnp.float32) + fb2[...],
        0.0).astype(jnp.bfloat16)
    logits = jnp.dot(h2, fw3[...], preferred_element_type=jnp.float32) + fb3[...]
    out[...] = logits[:, :10]


def kernel(x, conv1_w, conv1_b, conv2_w, conv2_b,
           fc_w1, fc_b1, fc_w2, fc_b2, fc_w3, fc_b3):
    B = x.shape[0]
    tb = _TB if B % _TB == 0 else B
    grid = B // tb

    bf = jnp.bfloat16
    A1 = jnp.asarray(_A1np, bf)
    C1 = jnp.asarray(_C1np, bf)
    B2 = jnp.asarray(_B2np, bf)
    O2 = jnp.asarray(_O2np, bf)
    A2 = jnp.asarray(_A2np, bf)
    CB1 = jnp.asarray(_CB1np, jnp.float32)
    CB2 = jnp.asarray(_CB2np, jnp.float32)

    x3d = x.reshape(B, 28, 28)

    res = pl.pallas_call(
        _net_kernel,
        out_shape=jax.ShapeDtypeStruct((B, 10), jnp.float32),
        grid=(grid,),
        in_specs=[
            pl.BlockSpec((tb, 28, 28), lambda i: (i, 0, 0)),
            pl.BlockSpec((9, 3, 176, 1024), lambda i: (0, 0, 0, 0)),
            pl.BlockSpec((6, 1024), lambda i: (0, 0)),
            pl.BlockSpec((2304, 54), lambda i: (0, 0)),
            pl.BlockSpec((128, 512), lambda i: (0, 0)),
            pl.BlockSpec((9, 2, 256, 512), lambda i: (0, 0, 0, 0)),
            pl.BlockSpec((8, 128), lambda i: (0, 0)),
            pl.BlockSpec((16, 128), lambda i: (0, 0)),
            pl.BlockSpec((9, 128), lambda i: (0, 0)),
            pl.BlockSpec((54, 128), lambda i: (0, 0)),
            pl.BlockSpec((1, 128), lambda i: (0, 0)),
            pl.BlockSpec((1, 128), lambda i: (0, 0)),
            pl.BlockSpec((784, 128), lambda i: (0, 0)),
            pl.BlockSpec((128, 128), lambda i: (0, 0)),
            pl.BlockSpec((128, 128), lambda i: (0, 0)),
            pl.BlockSpec((1, 128), lambda i: (0, 0)),
            pl.BlockSpec((1, 128), lambda i: (0, 0)),
            pl.BlockSpec((1, 128), lambda i: (0, 0)),
        ],
        out_specs=pl.BlockSpec((tb, 10), lambda i: (i, 0)),
        scratch_shapes=[
            pltpu.VMEM((3, 176, 1024), bf),
            pltpu.VMEM((512, 512), bf),
            pltpu.VMEM((4, 256, 128), bf),
            pltpu.VMEM((1, 128), jnp.float32),
            pltpu.VMEM((1, 128), jnp.float32),
        ],
        compiler_params=pltpu.CompilerParams(
            dimension_semantics=("arbitrary",)),
    )(x3d, A1, C1, B2, O2, A2, CB1, CB2,
      conv1_w, conv2_w, conv1_b, conv2_b, fc_w1, fc_w2, fc_w3,
      fc_b1, fc_b2, fc_b3)

    return res
